# baseline (speedup 1.0000x reference)
import functools

import jax
import jax.numpy as jnp
from jax import lax
from jax.experimental import pallas as pl
from jax.experimental.pallas import tpu as pltpu
from jax.experimental.pallas import tpu_sc as plsc

N = 10000
E = 160000
C = 256
BLK = 1000

NPAD = 10240
RPT = 320
TRASH = RPT
EPT = E // 16
EB = 2000
NB = EPT // EB
NBAT = E // EB
G = 32
CB = EB + 5 * G
DSLC = NPAD // 16


def _rsqrt16(d):
    i = plsc.bitcast(d, jnp.int32)
    i = jnp.int32(0x5F3759DF) - (i >> 1)
    y = plsc.bitcast(i, jnp.float32)
    for _ in range(3):
        y = y * (1.5 - 0.5 * d * y * y)
    return y


def _sc_body(src_hbm, dst_hbm, ew_hbm, x_hbm,
             agg_hbm, dinv_hbm,
             eb_src0, eb_dst0, eb_ew0, eb_src1, eb_dst1, eb_ew1,
             c_src, c_dst, c_ew, nbuf,
             dinv_loc, rows_gA, rows_gB, acc,
             deg_sh, dinv_sh,
             sem_gA, sem_gB, semA0, semA1, semA2, semB0, semB1, semB2):
    c = lax.axis_index("c")
    s = lax.axis_index("s")
    w = c * 16 + s
    base = w * RPT
    tile_e0 = s * EPT

    zf = jnp.zeros((16,), jnp.float32)
    zi = jnp.zeros((16,), jnp.int32)
    ti = jnp.full((16,), TRASH, jnp.int32)
    ones = jnp.ones((16,), jnp.float32)
    iota16 = lax.iota(jnp.int32, 16)

    for r in range(DSLC // 16):
        eb_ew0[pl.ds(16 * r, 16)] = zf
    pltpu.sync_copy(eb_ew0.at[pl.ds(0, DSLC)], deg_sh.at[pl.ds(s * DSLC, DSLC)])
    plsc.subcore_barrier()

    def dbody(b, _):
        e0 = tile_e0 + b * EB
        pltpu.sync_copy(dst_hbm.at[pl.ds(e0, EB)], eb_dst0)
        pltpu.sync_copy(ew_hbm.at[pl.ds(e0, EB)], eb_ew0)
        pltpu.sync_copy(eb_ew0, deg_sh.at[eb_dst0], add=True)
        return 0

    lax.fori_loop(0, NB, dbody, 0)
    plsc.subcore_barrier()

    pltpu.sync_copy(deg_sh.at[pl.ds(s * DSLC, DSLC)], eb_ew0.at[pl.ds(0, DSLC)])
    for i in range(DSLC // 16):
        d = eb_ew0[pl.ds(16 * i, 16)] + 1.0
        eb_ew0[pl.ds(16 * i, 16)] = _rsqrt16(d)
    pltpu.sync_copy(eb_ew0.at[pl.ds(0, DSLC)], dinv_sh.at[pl.ds(s * DSLC, DSLC)])

    @pl.when(c == 0)
    def _():
        pltpu.sync_copy(eb_ew0.at[pl.ds(0, DSLC)],
                        dinv_hbm.at[pl.ds(s * DSLC, DSLC)])

    plsc.subcore_barrier()
    pltpu.sync_copy(dinv_sh, dinv_loc.at[pl.ds(0, NPAD)])
    dinv_loc[pl.ds(NPAD, 16)] = ones

    def zbody(i, _):
        for k in range(C // 16):
            acc[i, pl.ds(16 * k, 16)] = zf
        return 0

    lax.fori_loop(0, RPT + 1, zbody, 0)

    ebs = [(eb_src0, eb_dst0, eb_ew0, semA0, semA1, semA2),
           (eb_src1, eb_dst1, eb_ew1, semB0, semB1, semB2)]

    def issue(b, p):
        e0 = b * EB
        S = ebs[p]
        pltpu.async_copy(src_hbm.at[pl.ds(e0, EB)], S[0], S[3])
        pltpu.async_copy(dst_hbm.at[pl.ds(e0, EB)], S[1], S[4])
        pltpu.async_copy(ew_hbm.at[pl.ds(e0, EB)], S[2], S[5])

    def waitp(p):
        S = ebs[p]
        pltpu.make_async_copy(src_hbm.at[pl.ds(0, EB)], S[0], S[3]).wait()
        pltpu.make_async_copy(dst_hbm.at[pl.ds(0, EB)], S[1], S[4]).wait()
        pltpu.make_async_copy(ew_hbm.at[pl.ds(0, EB)], S[2], S[5]).wait()

    def process(p):
        S = ebs[p]

        def pbody(i, _):
            c_ew[pl.ds(16 * i, 16)] = zf
            return 0

        lax.fori_loop(0, CB // 16, pbody, 0)

        def cbody(i, k):
            sv = S[0][pl.ds(16 * i, 16)]
            dv = S[1][pl.ds(16 * i, 16)]
            wv = S[2][pl.ds(16 * i, 16)]
            m = (dv >= base) & (dv < base + RPT)
            plsc.store_compressed(c_src.at[pl.ds(k, 16)], sv, mask=m)
            plsc.store_compressed(c_dst.at[pl.ds(k, 16)], dv - base, mask=m)
            plsc.store_compressed(c_ew.at[pl.ds(k, 16)], wv, mask=m)
            return k + jnp.sum(m.astype(jnp.int32))

        kfin = lax.fori_loop(0, EB // 16, cbody, jnp.int32(0))
        nsub = (kfin + (G - 1)) // G

        rgs = [(rows_gA, sem_gA), (rows_gB, sem_gB)]

        def rissue(j, q):
            R = rgs[q]
            pltpu.async_copy(x_hbm.at[c_src.at[pl.ds(G * j, G)]], R[0], R[1])

        def rwait(q):
            R = rgs[q]
            pltpu.make_async_copy(
                x_hbm.at[c_src.at[pl.ds(0, G)]], R[0], R[1]).wait()

        def raccum(j, q):
            R = rgs[q]
            for t in range(G // 16):
                off = G * j + 16 * t
                sv = c_src[pl.ds(off, 16)]
                dlv = c_dst[pl.ds(off, 16)]
                wv = c_ew[pl.ds(off, 16)]
                nv = wv * plsc.load_gather(dinv_loc, [sv]) \
                        * plsc.load_gather(dinv_loc, [dlv + base])
                for i in range(16):
                    r = dlv[i]
                    bi = jnp.full((16,), nv[i])
                    row = 16 * t + i
                    for kk in range(C // 16):
                        plsc.addupdate(
                            acc.at[r, pl.ds(16 * kk, 16)],
                            R[0][row, pl.ds(16 * kk, 16)] * bi)

        rissue(0, 0)

        def gpair(j2, _):
            j0 = 2 * j2
            rissue(j0 + 1, 1)
            rwait(0)
            raccum(j0, 0)
            rissue(j0 + 2, 0)
            rwait(1)
            raccum(j0 + 1, 1)
            return 0

        lax.fori_loop(0, (nsub + 1) // 2, gpair, 0)
        rwait(0)

    def p0body(i, _):
        c_src[pl.ds(16 * i, 16)] = zi
        c_dst[pl.ds(16 * i, 16)] = ti
        return 0

    lax.fori_loop(0, CB // 16, p0body, 0)

    issue(0, 0)

    def pair(i, _):
        b0 = 2 * i
        issue(b0 + 1, 1)
        waitp(0)
        process(0)

        @pl.when(b0 + 2 < NBAT)
        def _():
            issue(b0 + 2, 0)

        waitp(1)
        process(1)
        return 0

    lax.fori_loop(0, NBAT // 2, pair, 0)

    pltpu.sync_copy(acc.at[pl.ds(0, RPT)], agg_hbm.at[pl.ds(base, RPT)])


@functools.partial(jax.jit, donate_argnums=())
def _sc_aggregate(src, dst, ew, x):
    mesh = plsc.VectorSubcoreMesh(core_axis_name="c", subcore_axis_name="s")
    f32 = jnp.float32
    i32 = jnp.int32
    run = pl.kernel(
        _sc_body,
        out_type=[jax.ShapeDtypeStruct((NPAD, C), f32),
                  jax.ShapeDtypeStruct((NPAD,), f32)],
        mesh=mesh,
        scratch_types=[
            pltpu.VMEM((EB,), i32),
            pltpu.VMEM((EB,), i32),
            pltpu.VMEM((EB,), f32),
            pltpu.VMEM((EB,), i32),
            pltpu.VMEM((EB,), i32),
            pltpu.VMEM((EB,), f32),
            pltpu.VMEM((CB,), i32),
            pltpu.VMEM((CB,), i32),
            pltpu.VMEM((CB,), f32),
            pltpu.VMEM((16,), f32),
            pltpu.VMEM((NPAD + 16,), f32),
            pltpu.VMEM((G, C), f32),
            pltpu.VMEM((G, C), f32),
            pltpu.VMEM((RPT + 1, C), f32),
            pltpu.VMEM_SHARED((NPAD,), f32),
            pltpu.VMEM_SHARED((NPAD,), f32),
            pltpu.SemaphoreType.DMA,
            pltpu.SemaphoreType.DMA,
            pltpu.SemaphoreType.DMA,
            pltpu.SemaphoreType.DMA,
            pltpu.SemaphoreType.DMA,
            pltpu.SemaphoreType.DMA,
            pltpu.SemaphoreType.DMA,
            pltpu.SemaphoreType.DMA,
        ],
        compiler_params=pltpu.CompilerParams(needs_layout_passes=False),
    )
    return run(src, dst, ew, x)


def _dense_body(a_ref, h_ref, x_ref, d_ref,
                wz_ref, wr_ref, wh_ref,
                lz_ref, lr_ref, lh_ref,
                wlin_ref, bvec_ref,
                y_ref, hn_ref):
    bz = bvec_ref[0, :]
    br = bvec_ref[1, :]
    bh = bvec_ref[2, :]
    lbz = bvec_ref[3, :]
    lbr = bvec_ref[4, :]
    lbh = bvec_ref[5, :]
    blin = bvec_ref[6, :]

    dinv = d_ref[:, :]
    a = a_ref[:, :] + x_ref[:, :] * (dinv * dinv)
    h = h_ref[:, :]

    f32 = jnp.float32
    cz = jnp.dot(a, wz_ref[:, :], preferred_element_type=f32) + bz
    cr = jnp.dot(a, wr_ref[:, :], preferred_element_type=f32) + br
    ch = jnp.dot(a, wh_ref[:, :], preferred_element_type=f32) + bh

    z = jax.nn.sigmoid(
        jnp.dot(cz, lz_ref[:C, :], preferred_element_type=f32)
        + jnp.dot(h, lz_ref[C:, :], preferred_element_type=f32) + lbz)
    r = jax.nn.sigmoid(
        jnp.dot(cr, lr_ref[:C, :], preferred_element_type=f32)
        + jnp.dot(h, lr_ref[C:, :], preferred_element_type=f32) + lbr)
    ht = jnp.tanh(
        jnp.dot(ch, lh_ref[:C, :], preferred_element_type=f32)
        + jnp.dot(h * r, lh_ref[C:, :], preferred_element_type=f32) + lbh)
    hn = z * h + (1.0 - z) * ht
    y = jnp.dot(jax.nn.relu(hn), wlin_ref[:, :], preferred_element_type=f32) + blin
    y_ref[:, :] = y
    hn_ref[:, :] = hn


def _dense_gru(agg, h, x, dinv, Wz, Wr, Wh, Lz, Lr, Lh, Wlin, bvec):
    grid = (N // BLK,)
    blk_spec = pl.BlockSpec((BLK, C), lambda i: (i, 0))
    col_spec = pl.BlockSpec((BLK, 1), lambda i: (i, 0))
    full = lambda shape: pl.BlockSpec(shape, lambda i: (0, 0))
    return pl.pallas_call(
        _dense_body,
        grid=grid,
        in_specs=[blk_spec, blk_spec, blk_spec, col_spec,
                  full((C, C)), full((C, C)), full((C, C)),
                  full((2 * C, C)), full((2 * C, C)), full((2 * C, C)),
                  full((C, C)), full((7, C))],
        out_specs=[blk_spec, blk_spec],
        out_shape=[jax.ShapeDtypeStruct((N, C), jnp.float32),
                   jax.ShapeDtypeStruct((N, C), jnp.float32)],
    )(agg, h, x, dinv, Wz, Wr, Wh, Lz, Lr, Lh, Wlin, bvec)


def kernel(x, edge_index, edge_weight, prev_hidden_state,
           Wz, bz, Lz, lbz, Wr, br, Lr, lbr, Wh, bh, Lh, lbh, Wlin, blin):
    src, dst = edge_index[0], edge_index[1]
    agg_pad, dinv_pad = _sc_aggregate(src, dst, edge_weight, x)
    agg = agg_pad[:N]
    dinv = dinv_pad[:N]
    bvec = jnp.stack([bz, br, bh, lbz, lbr, lbh, blin])
    y, hn = _dense_gru(agg, prev_hidden_state, x, dinv[:, None],
                       Wz, Wr, Wh, Lz, Lr, Lh, Wlin, bvec)
    return (y, hn)


# speedup vs baseline: 1.9207x; 1.9207x over previous
import functools

import jax
import jax.numpy as jnp
from jax import lax
from jax.experimental import pallas as pl
from jax.experimental.pallas import tpu as pltpu
from jax.experimental.pallas import tpu_sc as plsc

N = 10000
E = 160000
C = 256
BLK = 1000

NPAD = 10240
RPT = 320
TRASH = RPT
EPT = E // 16
EB = 2000
NB = EPT // EB
NBAT = E // EB
G = 64
CB = EB + G
DSLC = NPAD // 16


def _rsqrt16(d):
    i = plsc.bitcast(d, jnp.int32)
    i = jnp.int32(0x5F3759DF) - (i >> 1)
    y = plsc.bitcast(i, jnp.float32)
    for _ in range(3):
        y = y * (1.5 - 0.5 * d * y * y)
    return y


def _sc_body(src_hbm, dst_hbm, ew_hbm, x_hbm,
             agg_hbm, dinv_hbm,
             eb_src0, eb_dst0, eb_ew0, eb_src1, eb_dst1, eb_ew1,
             c_src, c_dst, c_ew, nbuf,
             dinv_loc, rows_gA, rows_gB, acc,
             deg_sh, dinv_sh,
             sem_gA, sem_gB, semA0, semA1, semA2, semB0, semB1, semB2):
    c = lax.axis_index("c")
    s = lax.axis_index("s")
    w = c * 16 + s
    base = w * RPT
    tile_e0 = s * EPT

    zf = jnp.zeros((16,), jnp.float32)
    zi = jnp.zeros((16,), jnp.int32)
    ti = jnp.full((16,), TRASH, jnp.int32)
    ones = jnp.ones((16,), jnp.float32)
    iota16 = lax.iota(jnp.int32, 16)

    for r in range(DSLC // 16):
        eb_ew0[pl.ds(16 * r, 16)] = zf
    pltpu.sync_copy(eb_ew0.at[pl.ds(0, DSLC)], deg_sh.at[pl.ds(s * DSLC, DSLC)])
    plsc.subcore_barrier()

    def dbody(b, _):
        e0 = tile_e0 + b * EB
        pltpu.sync_copy(dst_hbm.at[pl.ds(e0, EB)], eb_dst0)
        pltpu.sync_copy(ew_hbm.at[pl.ds(e0, EB)], eb_ew0)
        pltpu.sync_copy(eb_ew0, deg_sh.at[eb_dst0], add=True)
        return 0

    lax.fori_loop(0, NB, dbody, 0)
    plsc.subcore_barrier()

    pltpu.sync_copy(deg_sh.at[pl.ds(s * DSLC, DSLC)], eb_ew0.at[pl.ds(0, DSLC)])
    for i in range(DSLC // 16):
        d = eb_ew0[pl.ds(16 * i, 16)] + 1.0
        eb_ew0[pl.ds(16 * i, 16)] = _rsqrt16(d)
    pltpu.sync_copy(eb_ew0.at[pl.ds(0, DSLC)], dinv_sh.at[pl.ds(s * DSLC, DSLC)])

    @pl.when(c == 0)
    def _():
        pltpu.sync_copy(eb_ew0.at[pl.ds(0, DSLC)],
                        dinv_hbm.at[pl.ds(s * DSLC, DSLC)])

    plsc.subcore_barrier()
    pltpu.sync_copy(dinv_sh, dinv_loc.at[pl.ds(0, NPAD)])
    dinv_loc[pl.ds(NPAD, 16)] = ones

    def zbody(i, _):
        for k in range(C // 16):
            acc[i, pl.ds(16 * k, 16)] = zf
        return 0

    lax.fori_loop(0, RPT + 1, zbody, 0)

    ebs = [(eb_src0, eb_dst0, eb_ew0, semA0, semA1, semA2),
           (eb_src1, eb_dst1, eb_ew1, semB0, semB1, semB2)]

    def issue(b, p):
        e0 = b * EB
        S = ebs[p]
        pltpu.async_copy(src_hbm.at[pl.ds(e0, EB)], S[0], S[3])
        pltpu.async_copy(dst_hbm.at[pl.ds(e0, EB)], S[1], S[4])
        pltpu.async_copy(ew_hbm.at[pl.ds(e0, EB)], S[2], S[5])

    def waitp(p):
        S = ebs[p]
        pltpu.make_async_copy(src_hbm.at[pl.ds(0, EB)], S[0], S[3]).wait()
        pltpu.make_async_copy(dst_hbm.at[pl.ds(0, EB)], S[1], S[4]).wait()
        pltpu.make_async_copy(ew_hbm.at[pl.ds(0, EB)], S[2], S[5]).wait()

    def process(p):
        S = ebs[p]

        def pbody(i, _):
            c_ew[pl.ds(16 * i, 16)] = zf
            return 0

        lax.fori_loop(0, CB // 16, pbody, 0)

        def cbody(i, k):
            sv = S[0][pl.ds(16 * i, 16)]
            dv = S[1][pl.ds(16 * i, 16)]
            wv = S[2][pl.ds(16 * i, 16)]
            m = (dv >= base) & (dv < base + RPT)
            plsc.store_compressed(c_src.at[pl.ds(k, 16)], sv, mask=m)
            plsc.store_compressed(c_dst.at[pl.ds(k, 16)], dv - base, mask=m)
            plsc.store_compressed(c_ew.at[pl.ds(k, 16)], wv, mask=m)
            return k + jnp.sum(m.astype(jnp.int32))

        kfin = lax.fori_loop(0, EB // 16, cbody, jnp.int32(0))
        nsub = (kfin + (G - 1)) // G

        def gbody(j, _):
            pltpu.async_copy(
                x_hbm.at[c_src.at[pl.ds(G * j, G)]], rows_gA, sem_gA).wait()
            for t in range(G // 16):
                off = G * j + 16 * t
                sv = c_src[pl.ds(off, 16)]
                dlv = c_dst[pl.ds(off, 16)]
                wv = c_ew[pl.ds(off, 16)]
                nv = wv * plsc.load_gather(dinv_loc, [sv]) \
                        * plsc.load_gather(dinv_loc, [dlv + base])
                for i in range(16):
                    r = dlv[i]
                    bi = jnp.full((16,), nv[i])
                    row = 16 * t + i
                    for kk in range(C // 16):
                        plsc.addupdate(
                            acc.at[r, pl.ds(16 * kk, 16)],
                            rows_gA[row, pl.ds(16 * kk, 16)] * bi)
            return 0

        lax.fori_loop(0, nsub, gbody, 0)

    def p0body(i, _):
        c_src[pl.ds(16 * i, 16)] = zi
        c_dst[pl.ds(16 * i, 16)] = ti
        return 0

    lax.fori_loop(0, CB // 16, p0body, 0)

    issue(0, 0)

    def pair(i, _):
        b0 = 2 * i
        issue(b0 + 1, 1)
        waitp(0)
        process(0)

        @pl.when(b0 + 2 < NBAT)
        def _():
            issue(b0 + 2, 0)

        waitp(1)
        process(1)
        return 0

    lax.fori_loop(0, NBAT // 2, pair, 0)

    pltpu.sync_copy(acc.at[pl.ds(0, RPT)], agg_hbm.at[pl.ds(base, RPT)])


@functools.partial(jax.jit, donate_argnums=())
def _sc_aggregate(src, dst, ew, x):
    mesh = plsc.VectorSubcoreMesh(core_axis_name="c", subcore_axis_name="s")
    f32 = jnp.float32
    i32 = jnp.int32
    run = pl.kernel(
        _sc_body,
        out_type=[jax.ShapeDtypeStruct((NPAD, C), f32),
                  jax.ShapeDtypeStruct((NPAD,), f32)],
        mesh=mesh,
        scratch_types=[
            pltpu.VMEM((EB,), i32),
            pltpu.VMEM((EB,), i32),
            pltpu.VMEM((EB,), f32),
            pltpu.VMEM((EB,), i32),
            pltpu.VMEM((EB,), i32),
            pltpu.VMEM((EB,), f32),
            pltpu.VMEM((CB,), i32),
            pltpu.VMEM((CB,), i32),
            pltpu.VMEM((CB,), f32),
            pltpu.VMEM((16,), f32),
            pltpu.VMEM((NPAD + 16,), f32),
            pltpu.VMEM((G, C), f32),
            pltpu.VMEM((G, C), f32),
            pltpu.VMEM((RPT + 1, C), f32),
            pltpu.VMEM_SHARED((NPAD,), f32),
            pltpu.VMEM_SHARED((NPAD,), f32),
            pltpu.SemaphoreType.DMA,
            pltpu.SemaphoreType.DMA,
            pltpu.SemaphoreType.DMA,
            pltpu.SemaphoreType.DMA,
            pltpu.SemaphoreType.DMA,
            pltpu.SemaphoreType.DMA,
            pltpu.SemaphoreType.DMA,
            pltpu.SemaphoreType.DMA,
        ],
        compiler_params=pltpu.CompilerParams(needs_layout_passes=False),
    )
    return run(src, dst, ew, x)


def _dense_body(a_ref, h_ref, x_ref, d_ref,
                wz_ref, wr_ref, wh_ref,
                lz_ref, lr_ref, lh_ref,
                wlin_ref, bvec_ref,
                y_ref, hn_ref):
    bz = bvec_ref[0, :]
    br = bvec_ref[1, :]
    bh = bvec_ref[2, :]
    lbz = bvec_ref[3, :]
    lbr = bvec_ref[4, :]
    lbh = bvec_ref[5, :]
    blin = bvec_ref[6, :]

    dinv = d_ref[:, :]
    a = a_ref[:, :] + x_ref[:, :] * (dinv * dinv)
    h = h_ref[:, :]

    f32 = jnp.float32
    cz = jnp.dot(a, wz_ref[:, :], preferred_element_type=f32) + bz
    cr = jnp.dot(a, wr_ref[:, :], preferred_element_type=f32) + br
    ch = jnp.dot(a, wh_ref[:, :], preferred_element_type=f32) + bh

    z = jax.nn.sigmoid(
        jnp.dot(cz, lz_ref[:C, :], preferred_element_type=f32)
        + jnp.dot(h, lz_ref[C:, :], preferred_element_type=f32) + lbz)
    r = jax.nn.sigmoid(
        jnp.dot(cr, lr_ref[:C, :], preferred_element_type=f32)
        + jnp.dot(h, lr_ref[C:, :], preferred_element_type=f32) + lbr)
    ht = jnp.tanh(
        jnp.dot(ch, lh_ref[:C, :], preferred_element_type=f32)
        + jnp.dot(h * r, lh_ref[C:, :], preferred_element_type=f32) + lbh)
    hn = z * h + (1.0 - z) * ht
    y = jnp.dot(jax.nn.relu(hn), wlin_ref[:, :], preferred_element_type=f32) + blin
    y_ref[:, :] = y
    hn_ref[:, :] = hn


def _dense_gru(agg, h, x, dinv, Wz, Wr, Wh, Lz, Lr, Lh, Wlin, bvec):
    grid = (N // BLK,)
    blk_spec = pl.BlockSpec((BLK, C), lambda i: (i, 0))
    col_spec = pl.BlockSpec((BLK, 1), lambda i: (i, 0))
    full = lambda shape: pl.BlockSpec(shape, lambda i: (0, 0))
    return pl.pallas_call(
        _dense_body,
        grid=grid,
        in_specs=[blk_spec, blk_spec, blk_spec, col_spec,
                  full((C, C)), full((C, C)), full((C, C)),
                  full((2 * C, C)), full((2 * C, C)), full((2 * C, C)),
                  full((C, C)), full((7, C))],
        out_specs=[blk_spec, blk_spec],
        out_shape=[jax.ShapeDtypeStruct((N, C), jnp.float32),
                   jax.ShapeDtypeStruct((N, C), jnp.float32)],
    )(agg, h, x, dinv, Wz, Wr, Wh, Lz, Lr, Lh, Wlin, bvec)


def kernel(x, edge_index, edge_weight, prev_hidden_state,
           Wz, bz, Lz, lbz, Wr, br, Lr, lbr, Wh, bh, Lh, lbh, Wlin, blin):
    src, dst = edge_index[0], edge_index[1]
    agg_pad, dinv_pad = _sc_aggregate(src, dst, edge_weight, x)
    agg = agg_pad[:N]
    dinv = dinv_pad[:N]
    bvec = jnp.stack([bz, br, bh, lbz, lbr, lbh, blin])
    y, hn = _dense_gru(agg, prev_hidden_state, x, dinv[:, None],
                       Wz, Wr, Wh, Lz, Lr, Lh, Wlin, bvec)
    return (y, hn)
